# baseline (speedup 1.0000x reference)
import functools

import jax
import jax.numpy as jnp
from jax import lax
from jax.experimental import pallas as pl
from jax.experimental.pallas import tpu as pltpu
from jax.experimental.pallas import tpu_sc as plsc


def _get_positive_k(k, n):
    if k <= 0:
        return 0
    elif k < 1:
        return round(k * n)
    elif k > n:
        return int(n)
    else:
        return int(k)


def _make_sc_kernel(rows, n, kmax, nc, ns, lanes_n, rpw):
    mesh = plsc.VectorSubcoreMesh(core_axis_name="c", subcore_axis_name="s")
    nbkt = 256
    hist_words = nbkt * lanes_n

    @functools.partial(
        pl.kernel,
        out_type=jax.ShapeDtypeStruct((rows,), jnp.float32),
        mesh=mesh,
        compiler_params=pltpu.CompilerParams(needs_layout_passes=False),
        scratch_types=[
            pltpu.VMEM((n,), jnp.float32),
            pltpu.VMEM((hist_words,), jnp.int32),
            pltpu.VMEM((hist_words,), jnp.float32),
            pltpu.VMEM((rpw,), jnp.float32),
        ],
    )
    def sc_kernel(x_hbm, out_hbm, xbuf, cnt_ref, sum_ref, res_ref):
        wid = lax.axis_index("s") * nc + lax.axis_index("c")
        lanes = lax.iota(jnp.int32, lanes_n)
        ones = jnp.ones((lanes_n,), jnp.int32)
        zi = jnp.zeros((lanes_n,), jnp.int32)
        zf = jnp.zeros((lanes_n,), jnp.float32)

        def row_body(j, _carry):
            row = wid * rpw + j
            pltpu.sync_copy(x_hbm.at[row], xbuf)

            pfx = jnp.int32(0)
            k_rem = jnp.int32(kmax)
            sum_acc = jnp.float32(0.0)
            for lvl in range(4):
                shift = 24 - 8 * lvl

                def zbody(c, _):
                    for u in range(4):
                        off = (c * 4 + u) * lanes_n
                        cnt_ref[pl.ds(off, lanes_n)] = zi
                        sum_ref[pl.ds(off, lanes_n)] = zf
                    return 0

                lax.fori_loop(0, nbkt // 4, zbody, 0)

                def dbody(i, _, pfx=pfx):
                    for u in range(4):
                        xv = xbuf[pl.ds((i * 4 + u) * lanes_n, lanes_n)]
                        bv = lax.bitcast_convert_type(xv, jnp.int32)
                        mv = jnp.where(
                            bv >= 0, bv, -(bv & jnp.int32(0x7FFFFFFF))
                        )
                        if lvl == 0:
                            digit = (mv >> 24) + 128
                            idx = (digit << 4) + lanes
                            plsc.addupdate_scatter(cnt_ref, [idx], ones)
                            plsc.addupdate_scatter(sum_ref, [idx], xv)
                        else:
                            digit = (mv >> shift) & 0xFF
                            mask = (mv >> (shift + 8)) == pfx
                            idx = (digit << 4) + lanes
                            plsc.addupdate_scatter(
                                cnt_ref, [idx], ones, mask=mask
                            )
                            plsc.addupdate_scatter(
                                sum_ref, [idx], xv, mask=mask
                            )
                    return 0

                lax.fori_loop(0, n // lanes_n // 4, dbody, 0)

                def sbody(s, carry, k_rem=k_rem):
                    above_c, above_s, bkt, cnt_ab, sum_ab = carry
                    for u in range(2):
                        bb = nbkt - 1 - (s * 2 + u)
                        cv = cnt_ref[pl.ds(bb * lanes_n, lanes_n)]
                        sv = sum_ref[pl.ds(bb * lanes_n, lanes_n)]
                        tot_c = jnp.sum(cv)
                        tot_s = jnp.sum(sv)
                        new_above = above_c + tot_c
                        cross = (above_c < k_rem) & (new_above >= k_rem)
                        bkt = jnp.where(cross, bb, bkt)
                        cnt_ab = jnp.where(cross, above_c, cnt_ab)
                        sum_ab = jnp.where(cross, above_s, sum_ab)
                        above_c = new_above
                        above_s = above_s + tot_s
                    return above_c, above_s, bkt, cnt_ab, sum_ab

                init = (
                    jnp.int32(0),
                    jnp.float32(0.0),
                    jnp.int32(0),
                    jnp.int32(0),
                    jnp.float32(0.0),
                )
                _, _, bkt, cnt_ab, sum_ab = lax.fori_loop(
                    0, nbkt // 2, sbody, init
                )
                k_rem = k_rem - cnt_ab
                sum_acc = sum_acc + sum_ab
                pfx = (bkt - 128) if lvl == 0 else ((pfx << 8) + bkt)

            t_m = pfx
            t_b = jnp.where(t_m >= 0, t_m, (-t_m) | jnp.int32(-0x80000000))
            t_bv = jnp.broadcast_to(t_b, (lanes_n,))
            t_f = jnp.max(lax.bitcast_convert_type(t_bv, jnp.float32))
            total = sum_acc + k_rem.astype(jnp.float32) * t_f
            resv = jnp.broadcast_to(total * (1.0 / kmax), (lanes_n,))
            jidx = jnp.broadcast_to(j, (lanes_n,))
            plsc.store_scatter(res_ref, [jidx], resv, mask=lanes == 0)
            return 0

        lax.fori_loop(0, rpw, row_body, 0)
        base = pl.multiple_of(wid * rpw, 8)
        pltpu.sync_copy(res_ref, out_hbm.at[pl.ds(base, rpw)])

    return sc_kernel


def kernel(input):
    batch, chan, h, w = input.shape
    n = h * w
    kmax = _get_positive_k(0.2, n)
    rows = batch * chan
    info = plsc.get_sparse_core_info()
    nc, ns, lanes_n = info.num_cores, info.num_subcores, info.num_lanes
    nw = nc * ns
    rpw = rows // nw
    x = input.reshape(rows, n)
    out = _make_sc_kernel(rows, n, kmax, nc, ns, lanes_n, rpw)(x)
    return out.reshape(batch, chan)


# speedup vs baseline: 1.1963x; 1.1963x over previous
import functools

import jax
import jax.numpy as jnp
from jax import lax
from jax.experimental import pallas as pl
from jax.experimental.pallas import tpu as pltpu
from jax.experimental.pallas import tpu_sc as plsc

_MIN32 = -2147483648


def _get_positive_k(k, n):
    if k <= 0:
        return 0
    elif k < 1:
        return round(k * n)
    elif k > n:
        return int(n)
    else:
        return int(k)


def _make_sc_kernel(rows, n, kmax, nc, ns, lanes_n, rpw):
    mesh = plsc.VectorSubcoreMesh(core_axis_name="c", subcore_axis_name="s")
    nbkt = 256
    hist_words = nbkt * lanes_n
    n_chunks = n // lanes_n
    unroll = 8

    @functools.partial(
        pl.kernel,
        out_type=jax.ShapeDtypeStruct((rows,), jnp.float32),
        mesh=mesh,
        compiler_params=pltpu.CompilerParams(needs_layout_passes=False),
        scratch_types=[
            pltpu.VMEM((n,), jnp.float32),
            pltpu.VMEM((n,), jnp.float32),
            pltpu.VMEM((hist_words,), jnp.int32),
            pltpu.VMEM((rpw,), jnp.float32),
            pltpu.SemaphoreType.DMA,
            pltpu.SemaphoreType.DMA,
        ],
    )
    def sc_kernel(x_hbm, out_hbm, buf0, buf1, cnt_ref, res_ref, sem0, sem1):
        wid = lax.axis_index("s") * nc + lax.axis_index("c")
        lanes = lax.iota(jnp.int32, lanes_n)
        lane_base = lanes * nbkt
        lane_b128 = lane_base + 128
        ones = jnp.ones((lanes_n,), jnp.int32)
        zi = jnp.zeros((lanes_n,), jnp.int32)
        zf = jnp.zeros((lanes_n,), jnp.float32)
        row0 = wid * rpw
        bufs = (buf0, buf1)
        sems = (sem0, sem1)

        pltpu.async_copy(x_hbm.at[row0], buf0, sem0)

        def process(buf, jj):
            pfx = jnp.int32(0)
            k_rem = jnp.int32(kmax)
            for lvl in range(4):
                shift = 24 - 8 * lvl

                def zbody(c, _):
                    for u in range(unroll):
                        cnt_ref[pl.ds((c * unroll + u) * lanes_n, lanes_n)] = (
                            zi
                        )
                    return 0

                lax.fori_loop(0, hist_words // lanes_n // unroll, zbody, 0)

                if lvl == 0:

                    def dbody(i, _):
                        for u in range(unroll):
                            off = (i * unroll + u) * lanes_n
                            xv = buf[pl.ds(off, lanes_n)]
                            bv = lax.bitcast_convert_type(xv, jnp.int32)
                            mv = jnp.where(bv >= 0, bv, _MIN32 - bv)
                            buf[pl.ds(off, lanes_n)] = (
                                lax.bitcast_convert_type(mv, jnp.float32)
                            )
                            idx = lane_b128 + (mv >> 24)
                            plsc.addupdate_scatter(cnt_ref, [idx], ones)
                        return 0

                else:

                    def dbody(i, _, pfx=pfx):
                        for u in range(unroll):
                            off = (i * unroll + u) * lanes_n
                            mv = lax.bitcast_convert_type(
                                buf[pl.ds(off, lanes_n)], jnp.int32
                            )
                            mask = (mv >> (shift + 8)) == pfx
                            idx = lane_base + ((mv >> shift) & 0xFF)
                            plsc.addupdate_scatter(
                                cnt_ref, [idx], ones, mask=mask
                            )
                        return 0

                lax.fori_loop(0, n_chunks // unroll, dbody, 0)

                def sbody(s, carry, k_rem=k_rem):
                    above_c, bcnt_vec, cab_vec = carry
                    c = nbkt // lanes_n - 1 - s
                    v = cnt_ref[pl.ds(c * lanes_n, lanes_n)]
                    for l in range(1, lanes_n):
                        v = v + cnt_ref[pl.ds(l * nbkt + c * lanes_n, lanes_n)]
                    s_vec = lax.rev(plsc.cumsum(lax.rev(v, (0,))), (0,))
                    s_vec = s_vec + above_c
                    ge = s_vec >= k_rem
                    bcnt_vec = bcnt_vec + jnp.where(ge, 1, 0)
                    cab_vec = cab_vec + jnp.where(ge, 0, v)
                    return above_c + jnp.sum(v), bcnt_vec, cab_vec

                _, bcnt_vec, cab_vec = lax.fori_loop(
                    0, nbkt // lanes_n, sbody, (jnp.int32(0), zi, zi)
                )
                bkt = jnp.sum(bcnt_vec) - 1
                k_rem = k_rem - jnp.sum(cab_vec)
                pfx = (bkt - 128) if lvl == 0 else ((pfx << 8) + bkt)

            t_m = pfx

            def fbody(i, acc):
                for u in range(unroll):
                    off = (i * unroll + u) * lanes_n
                    mv = lax.bitcast_convert_type(
                        buf[pl.ds(off, lanes_n)], jnp.int32
                    )
                    xv = lax.bitcast_convert_type(
                        jnp.where(mv >= 0, mv, _MIN32 - mv), jnp.float32
                    )
                    acc = acc + jnp.where(mv > t_m, xv, 0.0)
                return acc

            acc = lax.fori_loop(0, n_chunks // unroll, fbody, zf)
            sum_gt = jnp.sum(acc)
            t_b = jnp.where(t_m >= 0, t_m, _MIN32 - t_m)
            t_f = jnp.max(
                lax.bitcast_convert_type(
                    jnp.broadcast_to(t_b, (lanes_n,)), jnp.float32
                )
            )
            total = sum_gt + k_rem.astype(jnp.float32) * t_f
            resv = jnp.broadcast_to(total * (1.0 / kmax), (lanes_n,))
            jidx = jnp.broadcast_to(jj, (lanes_n,))
            plsc.store_scatter(res_ref, [jidx], resv, mask=lanes == 0)

        def pair_body(p, _):
            for phase in range(2):
                jj = p * 2 + phase
                buf = bufs[phase]
                pltpu.make_async_copy(x_hbm.at[row0], buf, sems[phase]).wait()
                nxt = jnp.minimum(jj + 1, rpw - 1)
                pltpu.async_copy(
                    x_hbm.at[row0 + nxt], bufs[1 - phase], sems[1 - phase]
                )
                process(buf, jj)
            return 0

        lax.fori_loop(0, rpw // 2, pair_body, 0)
        pltpu.make_async_copy(x_hbm.at[row0], buf0, sem0).wait()
        base = pl.multiple_of(wid * rpw, 8)
        pltpu.sync_copy(res_ref, out_hbm.at[pl.ds(base, rpw)])

    return sc_kernel


def kernel(input):
    batch, chan, h, w = input.shape
    n = h * w
    kmax = _get_positive_k(0.2, n)
    rows = batch * chan
    info = plsc.get_sparse_core_info()
    nc, ns, lanes_n = info.num_cores, info.num_subcores, info.num_lanes
    nw = nc * ns
    rpw = rows // nw
    x = input.reshape(rows, n)
    out = _make_sc_kernel(rows, n, kmax, nc, ns, lanes_n, rpw)(x)
    return out.reshape(batch, chan)


# speedup vs baseline: 5.1979x; 4.3449x over previous
import functools

import jax
import jax.numpy as jnp
from jax import lax
from jax.experimental import pallas as pl
from jax.experimental.pallas import tpu as pltpu
from jax.experimental.pallas import tpu_sc as plsc

_MIN32 = -2147483648


def _get_positive_k(k, n):
    if k <= 0:
        return 0
    elif k < 1:
        return round(k * n)
    elif k > n:
        return int(n)
    else:
        return int(k)


def _make_sc_kernel(rows, n, kmax, nc, ns, lanes_n, rpw):
    mesh = plsc.VectorSubcoreMesh(core_axis_name="c", subcore_axis_name="s")
    nbkt = 256
    hist_words = nbkt * lanes_n
    n_chunks = n // lanes_n
    unroll = 8

    @functools.partial(
        pl.kernel,
        out_type=jax.ShapeDtypeStruct((rows,), jnp.float32),
        mesh=mesh,
        compiler_params=pltpu.CompilerParams(needs_layout_passes=False),
        scratch_types=[
            pltpu.VMEM((n,), jnp.float32),
            pltpu.VMEM((n,), jnp.float32),
            pltpu.VMEM((hist_words,), jnp.int32),
            pltpu.VMEM((rpw,), jnp.float32),
            pltpu.SemaphoreType.DMA,
            pltpu.SemaphoreType.DMA,
        ],
    )
    def sc_kernel(x_hbm, out_hbm, buf0, buf1, cnt_ref, res_ref, sem0, sem1):
        wid = lax.axis_index("s") * nc + lax.axis_index("c")
        lanes = lax.iota(jnp.int32, lanes_n)
        lane_p = lanes + 128 * lanes_n
        ones = jnp.ones((lanes_n,), jnp.int32)
        zi = jnp.zeros((lanes_n,), jnp.int32)
        zf = jnp.zeros((lanes_n,), jnp.float32)
        rot16 = [lanes * lanes_n + ((l + lanes) & (lanes_n - 1))
                 for l in range(lanes_n)]
        row0 = wid * rpw
        bufs = (buf0, buf1)
        sems = (sem0, sem1)

        pltpu.async_copy(x_hbm.at[row0], buf0, sem0)

        def process(buf, jj):
            pfx = jnp.int32(0)
            k_rem = jnp.int32(kmax)
            for lvl in range(4):
                shift = 24 - 8 * lvl

                @plsc.parallel_loop(0, hist_words // lanes_n, unroll=unroll)
                def _zero(c):
                    cnt_ref[pl.ds(c * lanes_n, lanes_n)] = zi

                if lvl == 0:

                    @plsc.parallel_loop(0, n_chunks, unroll=unroll)
                    def _data(i):
                        off = i * lanes_n
                        xv = buf[pl.ds(off, lanes_n)]
                        bv = lax.bitcast_convert_type(xv, jnp.int32)
                        mv = jnp.where(bv >= 0, bv, _MIN32 - bv)
                        buf[pl.ds(off, lanes_n)] = lax.bitcast_convert_type(
                            mv, jnp.float32
                        )
                        idx = ((mv >> 24) << 4) + lane_p
                        plsc.addupdate_scatter(cnt_ref, [idx], ones)

                else:
                    pfx_now = pfx

                    @plsc.parallel_loop(0, n_chunks, unroll=unroll)
                    def _data(i):
                        off = i * lanes_n
                        mv = lax.bitcast_convert_type(
                            buf[pl.ds(off, lanes_n)], jnp.int32
                        )
                        mask = (mv >> (shift + 8)) == pfx_now
                        idx = (((mv >> shift) & 0xFF) << 4) + lanes
                        plsc.addupdate_scatter(cnt_ref, [idx], ones, mask=mask)

                def sbody(s, carry, k_rem=k_rem):
                    above_c, bcnt_vec, cab_vec = carry
                    c = nbkt // lanes_n - 1 - s
                    c_off = c * nbkt
                    v = plsc.load_gather(cnt_ref, [rot16[0] + c_off])
                    for l in range(1, lanes_n):
                        v = v + plsc.load_gather(cnt_ref, [rot16[l] + c_off])
                    s_vec = lax.rev(plsc.cumsum(lax.rev(v, (0,))), (0,))
                    s_vec = s_vec + above_c
                    ge = s_vec >= k_rem
                    bcnt_vec = bcnt_vec + jnp.where(ge, 1, 0)
                    cab_vec = cab_vec + jnp.where(ge, 0, v)
                    return above_c + jnp.sum(v), bcnt_vec, cab_vec

                _, bcnt_vec, cab_vec = lax.fori_loop(
                    0, nbkt // lanes_n, sbody, (jnp.int32(0), zi, zi)
                )
                bkt = jnp.sum(bcnt_vec) - 1
                k_rem = k_rem - jnp.sum(cab_vec)
                pfx = (bkt - 128) if lvl == 0 else ((pfx << 8) + bkt)

            t_m = pfx

            @plsc.parallel_loop(0, n_chunks, unroll=unroll, carry=zf)
            def acc(i, a):
                mv = lax.bitcast_convert_type(
                    buf[pl.ds(i * lanes_n, lanes_n)], jnp.int32
                )
                xv = lax.bitcast_convert_type(
                    jnp.where(mv >= 0, mv, _MIN32 - mv), jnp.float32
                )
                return a + jnp.where(mv > t_m, xv, 0.0)

            sum_gt = jnp.sum(acc)
            t_b = jnp.where(t_m >= 0, t_m, _MIN32 - t_m)
            t_f = jnp.max(
                lax.bitcast_convert_type(
                    jnp.broadcast_to(t_b, (lanes_n,)), jnp.float32
                )
            )
            total = sum_gt + k_rem.astype(jnp.float32) * t_f
            resv = jnp.broadcast_to(total * (1.0 / kmax), (lanes_n,))
            jidx = jnp.broadcast_to(jj, (lanes_n,))
            plsc.store_scatter(res_ref, [jidx], resv, mask=lanes == 0)

        def pair_body(p, _):
            for phase in range(2):
                jj = p * 2 + phase
                buf = bufs[phase]
                pltpu.make_async_copy(x_hbm.at[row0], buf, sems[phase]).wait()
                nxt = jnp.minimum(jj + 1, rpw - 1)
                pltpu.async_copy(
                    x_hbm.at[row0 + nxt], bufs[1 - phase], sems[1 - phase]
                )
                process(buf, jj)
            return 0

        lax.fori_loop(0, rpw // 2, pair_body, 0)
        pltpu.make_async_copy(x_hbm.at[row0], buf0, sem0).wait()
        base = pl.multiple_of(wid * rpw, 8)
        pltpu.sync_copy(res_ref, out_hbm.at[pl.ds(base, rpw)])

    return sc_kernel


def kernel(input):
    batch, chan, h, w = input.shape
    n = h * w
    kmax = _get_positive_k(0.2, n)
    rows = batch * chan
    info = plsc.get_sparse_core_info()
    nc, ns, lanes_n = info.num_cores, info.num_subcores, info.num_lanes
    nw = nc * ns
    rpw = rows // nw
    x = input.reshape(rows, n)
    out = _make_sc_kernel(rows, n, kmax, nc, ns, lanes_n, rpw)(x)
    return out.reshape(batch, chan)


# speedup vs baseline: 6.7168x; 1.2922x over previous
import functools

import jax
import jax.numpy as jnp
from jax import lax
from jax.experimental import pallas as pl
from jax.experimental.pallas import tpu as pltpu
from jax.experimental.pallas import tpu_sc as plsc

_MIN32 = -2147483648


def _get_positive_k(k, n):
    if k <= 0:
        return 0
    elif k < 1:
        return round(k * n)
    elif k > n:
        return int(n)
    else:
        return int(k)


def _make_sc_kernel(batch, chan, h, w, kmax, nc, ns, lanes_n, rpw):
    mesh = plsc.VectorSubcoreMesh(core_axis_name="c", subcore_axis_name="s")
    rows = batch * chan
    nbkt = 256
    hist_words = nbkt * lanes_n
    wch = w // lanes_n
    unroll = 2

    @functools.partial(
        pl.kernel,
        out_type=jax.ShapeDtypeStruct((rows,), jnp.float32),
        mesh=mesh,
        compiler_params=pltpu.CompilerParams(needs_layout_passes=False),
        scratch_types=[
            pltpu.VMEM((h, w), jnp.float32),
            pltpu.VMEM((h, w), jnp.float32),
            pltpu.VMEM((hist_words,), jnp.int32),
            pltpu.VMEM((rpw,), jnp.float32),
            pltpu.SemaphoreType.DMA,
            pltpu.SemaphoreType.DMA,
        ],
    )
    def sc_kernel(x_hbm, out_hbm, buf0, buf1, cnt_ref, res_ref, sem0, sem1):
        wid = lax.axis_index("s") * nc + lax.axis_index("c")
        lanes = lax.iota(jnp.int32, lanes_n)
        lane_p = lanes + 128 * lanes_n
        ones = jnp.ones((lanes_n,), jnp.int32)
        zi = jnp.zeros((lanes_n,), jnp.int32)
        zf = jnp.zeros((lanes_n,), jnp.float32)
        rot16 = [lanes * lanes_n + ((l + lanes) & (lanes_n - 1))
                 for l in range(lanes_n)]
        row0 = wid * rpw
        bufs = (buf0, buf1)
        sems = (sem0, sem1)

        def start_dma(row, phase):
            bi = row // chan
            ci = row - bi * chan
            pltpu.async_copy(x_hbm.at[bi, ci], bufs[phase], sems[phase])

        start_dma(row0, 0)

        def process(buf, jj):
            pfx = jnp.int32(0)
            k_rem = jnp.int32(kmax)
            for lvl in range(4):
                shift = 24 - 8 * lvl

                @plsc.parallel_loop(0, hist_words // lanes_n, unroll=8)
                def _zero(c):
                    cnt_ref[pl.ds(c * lanes_n, lanes_n)] = zi

                if lvl == 0:

                    @plsc.parallel_loop(0, h, unroll=unroll)
                    def _data(r):
                        for cc in range(wch):
                            xv = buf[r, pl.ds(cc * lanes_n, lanes_n)]
                            bv = lax.bitcast_convert_type(xv, jnp.int32)
                            mv = jnp.where(bv >= 0, bv, _MIN32 - bv)
                            buf[r, pl.ds(cc * lanes_n, lanes_n)] = (
                                lax.bitcast_convert_type(mv, jnp.float32)
                            )
                            idx = ((mv >> 24) << 4) + lane_p
                            plsc.addupdate_scatter(cnt_ref, [idx], ones)

                else:
                    pfx_now = pfx

                    @plsc.parallel_loop(0, h, unroll=unroll)
                    def _data(r):
                        for cc in range(wch):
                            mv = lax.bitcast_convert_type(
                                buf[r, pl.ds(cc * lanes_n, lanes_n)], jnp.int32
                            )
                            mask = (mv >> (shift + 8)) == pfx_now
                            idx = (((mv >> shift) & 0xFF) << 4) + lanes
                            plsc.addupdate_scatter(
                                cnt_ref, [idx], ones, mask=mask
                            )

                def sbody(s, carry, k_rem=k_rem):
                    above_c, bcnt_vec, cab_vec = carry
                    c = nbkt // lanes_n - 1 - s
                    c_off = c * nbkt
                    v = plsc.load_gather(cnt_ref, [rot16[0] + c_off])
                    for l in range(1, lanes_n):
                        v = v + plsc.load_gather(cnt_ref, [rot16[l] + c_off])
                    s_vec = lax.rev(plsc.cumsum(lax.rev(v, (0,))), (0,))
                    s_vec = s_vec + above_c
                    ge = s_vec >= k_rem
                    bcnt_vec = bcnt_vec + jnp.where(ge, 1, 0)
                    cab_vec = cab_vec + jnp.where(ge, 0, v)
                    return above_c + jnp.sum(v), bcnt_vec, cab_vec

                _, bcnt_vec, cab_vec = lax.fori_loop(
                    0, nbkt // lanes_n, sbody, (jnp.int32(0), zi, zi)
                )
                bkt = jnp.sum(bcnt_vec) - 1
                k_rem = k_rem - jnp.sum(cab_vec)
                pfx = (bkt - 128) if lvl == 0 else ((pfx << 8) + bkt)

            t_m = pfx

            @plsc.parallel_loop(0, h, unroll=unroll, carry=zf)
            def acc(r, a):
                for cc in range(wch):
                    mv = lax.bitcast_convert_type(
                        buf[r, pl.ds(cc * lanes_n, lanes_n)], jnp.int32
                    )
                    xv = lax.bitcast_convert_type(
                        jnp.where(mv >= 0, mv, _MIN32 - mv), jnp.float32
                    )
                    a = a + jnp.where(mv > t_m, xv, 0.0)
                return a

            sum_gt = jnp.sum(acc)
            t_b = jnp.where(t_m >= 0, t_m, _MIN32 - t_m)
            t_f = jnp.max(
                lax.bitcast_convert_type(
                    jnp.broadcast_to(t_b, (lanes_n,)), jnp.float32
                )
            )
            total = sum_gt + k_rem.astype(jnp.float32) * t_f
            resv = jnp.broadcast_to(total * (1.0 / kmax), (lanes_n,))
            jidx = jnp.broadcast_to(jj, (lanes_n,))
            plsc.store_scatter(res_ref, [jidx], resv, mask=lanes == 0)

        def pair_body(p, _):
            for phase in range(2):
                jj = p * 2 + phase
                buf = bufs[phase]
                pltpu.make_async_copy(
                    x_hbm.at[0, 0], buf, sems[phase]
                ).wait()
                nxt = jnp.minimum(jj + 1, rpw - 1)
                start_dma(row0 + nxt, 1 - phase)
                process(buf, jj)
            return 0

        lax.fori_loop(0, rpw // 2, pair_body, 0)
        pltpu.make_async_copy(x_hbm.at[0, 0], buf0, sem0).wait()
        base = pl.multiple_of(wid * rpw, 8)
        pltpu.sync_copy(res_ref, out_hbm.at[pl.ds(base, rpw)])

    return sc_kernel


def kernel(input):
    batch, chan, h, w = input.shape
    n = h * w
    kmax = _get_positive_k(0.2, n)
    info = plsc.get_sparse_core_info()
    nc, ns, lanes_n = info.num_cores, info.num_subcores, info.num_lanes
    nw = nc * ns
    rpw = (batch * chan) // nw
    out = _make_sc_kernel(batch, chan, h, w, kmax, nc, ns, lanes_n, rpw)(input)
    return out.reshape(batch, chan)
